# Initial kernel scaffold; baseline (speedup 1.0000x reference)
import jax
import jax.numpy as jnp
from jax.experimental import pallas as pl


def kernel(x, edge_attr, W_e1, b_e1, W_e2, b_e2, W_n1, b_n1, W_n2, b_n2, edge_index):
    raise NotImplementedError("write your pallas kernel here")


# speedup vs baseline: 2.7917x; 2.7917x over previous
import functools
import jax
import jax.numpy as jnp
from jax import lax
from jax.experimental import pallas as pl
from jax.experimental.pallas import tpu as pltpu
from jax.experimental.pallas import tpu_sc as plsc

N_NODES = 10000
N_EDGES = 320000
NODE_DIM = 128
EDGE_DIM = 16
HIDDEN_DIM = 128

_NC = 2
_NS = 16
_NW = _NC * _NS
_CHUNK = 128
_NCHUNKS = N_EDGES // _CHUNK
_CPW = -(-_NCHUNKS // _NW)

def _mesh():
    return plsc.VectorSubcoreMesh(core_axis_name="c", subcore_axis_name="s")


def _proj_body(x_ref, wr_ref, wc_ref, xr_ref, xc_ref):
    x = x_ref[...]
    xr_ref[...] = jnp.dot(x, wr_ref[...], preferred_element_type=jnp.float32)
    xc_ref[...] = jnp.dot(x, wc_ref[...], preferred_element_type=jnp.float32)


def _proj(x, wr, wc):
    blk = 1000
    grid = (N_NODES // blk,)
    return pl.pallas_call(
        _proj_body,
        grid=grid,
        in_specs=[
            pl.BlockSpec((blk, NODE_DIM), lambda i: (i, 0)),
            pl.BlockSpec((NODE_DIM, HIDDEN_DIM), lambda i: (0, 0)),
            pl.BlockSpec((NODE_DIM, HIDDEN_DIM), lambda i: (0, 0)),
        ],
        out_specs=[
            pl.BlockSpec((blk, HIDDEN_DIM), lambda i: (i, 0)),
            pl.BlockSpec((blk, HIDDEN_DIM), lambda i: (i, 0)),
        ],
        out_shape=[
            jax.ShapeDtypeStruct((N_NODES, HIDDEN_DIM), jnp.float32),
            jax.ShapeDtypeStruct((N_NODES, HIDDEN_DIM), jnp.float32),
        ],
    )(x, wr, wc)


def _sc_gather_body(xr_hbm, xc_hbm, row_hbm, col_hbm, g_hbm,
                    ridx, cidx, rbuf, cbuf, sem_r, sem_c):
    c = lax.axis_index("c")
    s = lax.axis_index("s")
    w = s * _NC + c

    @pl.loop(0, _CPW)
    def _chunk_loop(i):
        cid = i * _NW + w

        @pl.when(cid < _NCHUNKS)
        def _():
            base = cid * _CHUNK
            pltpu.sync_copy(row_hbm.at[pl.ds(base, _CHUNK)], ridx)
            pltpu.sync_copy(col_hbm.at[pl.ds(base, _CHUNK)], cidx)
            cp_r = pltpu.async_copy(xr_hbm.at[ridx], rbuf, sem_r)
            cp_c = pltpu.async_copy(xc_hbm.at[cidx], cbuf, sem_c)
            cp_r.wait()
            cp_c.wait()

            @pl.loop(0, _CHUNK)
            def _row_loop(j):
                for k in range(HIDDEN_DIM // 16):
                    sl = pl.ds(k * 16, 16)
                    rbuf[j, sl] = rbuf[j, sl] + cbuf[j, sl]

            pltpu.sync_copy(rbuf, g_hbm.at[pl.ds(base, _CHUNK)])


def _sc_gather(xr, xc, row, col):
    f = pl.kernel(
        _sc_gather_body,
        out_type=jax.ShapeDtypeStruct((N_EDGES, HIDDEN_DIM), jnp.float32),
        mesh=_mesh(),
        scratch_types=[
            pltpu.VMEM((_CHUNK,), jnp.int32),
            pltpu.VMEM((_CHUNK,), jnp.int32),
            pltpu.VMEM((_CHUNK, HIDDEN_DIM), jnp.float32),
            pltpu.VMEM((_CHUNK, HIDDEN_DIM), jnp.float32),
            pltpu.SemaphoreType.DMA,
            pltpu.SemaphoreType.DMA,
        ],
    )
    return f(xr, xc, row, col)


def _edge_mlp_body(g_ref, ea_ref, we1e_ref, be1_ref, we2_ref, be2_ref, out_ref):
    pre = (g_ref[...]
           + jnp.dot(ea_ref[...], we1e_ref[...],
                     preferred_element_type=jnp.float32)
           + be1_ref[...])
    h = jnp.maximum(pre, 0.0)
    out_ref[...] = (jnp.dot(h, we2_ref[...], preferred_element_type=jnp.float32)
                    + be2_ref[...])


def _edge_mlp(g, ea, we1e, be1, we2, be2):
    blk = 2000
    grid = (N_EDGES // blk,)
    return pl.pallas_call(
        _edge_mlp_body,
        grid=grid,
        in_specs=[
            pl.BlockSpec((blk, HIDDEN_DIM), lambda i: (i, 0)),
            pl.BlockSpec((blk, EDGE_DIM), lambda i: (i, 0)),
            pl.BlockSpec((EDGE_DIM, HIDDEN_DIM), lambda i: (0, 0)),
            pl.BlockSpec((1, HIDDEN_DIM), lambda i: (0, 0)),
            pl.BlockSpec((HIDDEN_DIM, EDGE_DIM), lambda i: (0, 0)),
            pl.BlockSpec((1, EDGE_DIM), lambda i: (0, 0)),
        ],
        out_specs=pl.BlockSpec((blk, EDGE_DIM), lambda i: (i, 0)),
        out_shape=jax.ShapeDtypeStruct((N_EDGES, EDGE_DIM), jnp.float32),
    )(g, ea, we1e, be1, we2, be2)


_ZCH = 400
_NZCH = N_NODES // _ZCH


_RCH = 80
_NRCH = N_NODES // _RCH


def _sc_scatter_body(e_hbm, col_hbm, out_hbm, cidx, ecompact, ebuf, nidx,
                     nbuf, acc, sem):
    c = lax.axis_index("c")
    s = lax.axis_index("s")
    w = s * _NC + c

    @pl.loop(0, _CHUNK)
    def _ze(j):
        for k in range(8):
            ebuf[j, pl.ds(k * 16, 16)] = jnp.zeros((16,), jnp.float32)

    @pl.loop(0, _RCH)
    def _zn(j):
        for k in range(8):
            nbuf[j, pl.ds(k * 16, 16)] = jnp.zeros((16,), jnp.float32)

    @pl.loop(0, 8)
    def _zero_chunks(i):
        zc = i * _NS + s

        @pl.when(zc < _NRCH)
        def _():
            @pl.loop(0, _RCH // 16)
            def _zi(k):
                nidx[pl.ds(k * 16, 16)] = (
                    zc * _RCH + k * 16 + lax.iota(jnp.int32, 16))

            pltpu.sync_copy(nbuf, acc.at[nidx])

    plsc.subcore_barrier()

    @pl.loop(0, _CPW)
    def _chunk_loop(i):
        cid = i * _NW + w

        @pl.when(cid < _NCHUNKS)
        def _():
            base = cid * _CHUNK
            pltpu.sync_copy(col_hbm.at[pl.ds(base, _CHUNK)], cidx)
            pltpu.sync_copy(e_hbm.at[pl.ds(base, _CHUNK)], ecompact)

            @pl.loop(0, _CHUNK)
            def _exp(j):
                ebuf[j, pl.ds(0, 16)] = ecompact[j, :]

            pltpu.sync_copy(ebuf, acc.at[cidx], add=True)

    plsc.subcore_barrier()

    @pl.loop(0, 8)
    def _out_chunks(i):
        zc = i * _NS + s

        @pl.when(zc < _NRCH)
        def _():
            @pl.loop(0, _RCH // 16)
            def _gi(k):
                nidx[pl.ds(k * 16, 16)] = (
                    zc * _RCH + k * 16 + lax.iota(jnp.int32, 16))

            pltpu.async_copy(acc.at[nidx], nbuf, sem).wait()
            pltpu.sync_copy(
                nbuf, out_hbm.at[pl.ds(c * N_NODES + zc * _RCH, _RCH)])


def _sc_scatter(e_new, col):
    f = pl.kernel(
        _sc_scatter_body,
        out_type=jax.ShapeDtypeStruct((_NC * N_NODES, HIDDEN_DIM),
                                      jnp.float32),
        mesh=_mesh(),
        scratch_types=[
            pltpu.VMEM((_CHUNK,), jnp.int32),
            pltpu.VMEM((_CHUNK, EDGE_DIM), jnp.float32),
            pltpu.VMEM((_CHUNK, HIDDEN_DIM), jnp.float32),
            pltpu.VMEM((_RCH,), jnp.int32),
            pltpu.VMEM((_RCH, HIDDEN_DIM), jnp.float32),
            pltpu.VMEM_SHARED((N_NODES, HIDDEN_DIM), jnp.float32),
            pltpu.SemaphoreType.DMA,
        ],
    )
    out = f(e_new, col)
    return out.reshape(_NC, N_NODES, HIDDEN_DIM)[:, :, :EDGE_DIM]


def _node_mlp_body(x_ref, m_ref, wn1x_ref, wn1m_ref, bn1_ref, wn2_ref,
                   bn2_ref, out_ref):
    m = m_ref[0] + m_ref[1]
    pre = (jnp.dot(x_ref[...], wn1x_ref[...],
                   preferred_element_type=jnp.float32)
           + jnp.dot(m, wn1m_ref[...], preferred_element_type=jnp.float32)
           + bn1_ref[...])
    h = jnp.maximum(pre, 0.0)
    out_ref[...] = (jnp.dot(h, wn2_ref[...], preferred_element_type=jnp.float32)
                    + bn2_ref[...])


def _node_mlp(x, partials, wn1x, wn1m, bn1, wn2, bn2):
    blk = 1000
    grid = (N_NODES // blk,)
    return pl.pallas_call(
        _node_mlp_body,
        grid=grid,
        in_specs=[
            pl.BlockSpec((blk, NODE_DIM), lambda i: (i, 0)),
            pl.BlockSpec((_NC, blk, EDGE_DIM), lambda i: (0, i, 0)),
            pl.BlockSpec((NODE_DIM, HIDDEN_DIM), lambda i: (0, 0)),
            pl.BlockSpec((EDGE_DIM, HIDDEN_DIM), lambda i: (0, 0)),
            pl.BlockSpec((1, HIDDEN_DIM), lambda i: (0, 0)),
            pl.BlockSpec((HIDDEN_DIM, NODE_DIM), lambda i: (0, 0)),
            pl.BlockSpec((1, NODE_DIM), lambda i: (0, 0)),
        ],
        out_specs=pl.BlockSpec((blk, NODE_DIM), lambda i: (i, 0)),
        out_shape=jax.ShapeDtypeStruct((N_NODES, NODE_DIM), jnp.float32),
    )(x, partials, wn1x, wn1m, bn1, wn2, bn2)


def kernel(x, edge_attr, W_e1, b_e1, W_e2, b_e2, W_n1, b_n1, W_n2, b_n2,
           edge_index):
    row = edge_index[0].astype(jnp.int32)
    col = edge_index[1].astype(jnp.int32)
    we1e = W_e1[:EDGE_DIM]
    wr = W_e1[EDGE_DIM:EDGE_DIM + NODE_DIM]
    wc = W_e1[EDGE_DIM + NODE_DIM:]
    wn1x = W_n1[:NODE_DIM]
    wn1m = W_n1[NODE_DIM:]

    xr, xc = _proj(x, wr, wc)
    g = _sc_gather(xr, xc, row, col)
    e_new = _edge_mlp(g, edge_attr, we1e, b_e1.reshape(1, -1),
                      W_e2, b_e2.reshape(1, -1))
    partials = _sc_scatter(e_new, col)
    x_new = _node_mlp(x, partials, wn1x, wn1m, b_n1.reshape(1, -1),
                      W_n2, b_n2.reshape(1, -1))
    return (x_new, e_new)


# speedup vs baseline: 3.3610x; 1.2039x over previous
import functools
import jax
import jax.numpy as jnp
from jax import lax
from jax.experimental import pallas as pl
from jax.experimental.pallas import tpu as pltpu
from jax.experimental.pallas import tpu_sc as plsc

N_NODES = 10000
N_EDGES = 320000
NODE_DIM = 128
EDGE_DIM = 16
HIDDEN_DIM = 128

_NC = 2
_NS = 16
_NW = _NC * _NS
_CHUNK = 128
_NCHUNKS = N_EDGES // _CHUNK
_CPW = -(-_NCHUNKS // _NW)

def _mesh():
    return plsc.VectorSubcoreMesh(core_axis_name="c", subcore_axis_name="s")


def _proj_body(x_ref, wr_ref, wc_ref, xr_ref, xc_ref):
    x = x_ref[...]
    xr_ref[...] = jnp.dot(x, wr_ref[...], preferred_element_type=jnp.float32)
    xc_ref[...] = jnp.dot(x, wc_ref[...], preferred_element_type=jnp.float32)


def _proj(x, wr, wc):
    blk = 1000
    grid = (N_NODES // blk,)
    return pl.pallas_call(
        _proj_body,
        grid=grid,
        in_specs=[
            pl.BlockSpec((blk, NODE_DIM), lambda i: (i, 0)),
            pl.BlockSpec((NODE_DIM, HIDDEN_DIM), lambda i: (0, 0)),
            pl.BlockSpec((NODE_DIM, HIDDEN_DIM), lambda i: (0, 0)),
        ],
        out_specs=[
            pl.BlockSpec((blk, HIDDEN_DIM), lambda i: (i, 0)),
            pl.BlockSpec((blk, HIDDEN_DIM), lambda i: (i, 0)),
        ],
        out_shape=[
            jax.ShapeDtypeStruct((N_NODES, HIDDEN_DIM), jnp.float32),
            jax.ShapeDtypeStruct((N_NODES, HIDDEN_DIM), jnp.float32),
        ],
    )(x, wr, wc)


_EPT = N_EDGES // _NW
_GCH = 80
_GNCH = _EPT // _GCH


def _sc_gather_body(xr_hbm, xc_hbm, row_hbm, col_hbm, g_hbm,
                    rowv, colv, rbuf0, rbuf1, cbuf0, cbuf1,
                    sem_r0, sem_r1, sem_c0, sem_c1, sem_o0, sem_o1):
    c = lax.axis_index("c")
    s = lax.axis_index("s")
    w = s * _NC + c
    ebase = w * _EPT
    rbufs = (rbuf0, rbuf1)
    cbufs = (cbuf0, cbuf1)
    sems_r = (sem_r0, sem_r1)
    sems_c = (sem_c0, sem_c1)
    sems_o = (sem_o0, sem_o1)

    pltpu.sync_copy(row_hbm.at[pl.ds(ebase, _EPT)], rowv)
    pltpu.sync_copy(col_hbm.at[pl.ds(ebase, _EPT)], colv)

    pltpu.async_copy(xr_hbm.at[rowv.at[pl.ds(0, _GCH)]], rbuf0, sem_r0)
    pltpu.async_copy(xc_hbm.at[colv.at[pl.ds(0, _GCH)]], cbuf0, sem_c0)

    @pl.loop(0, (_GNCH + 1) // 2)
    def _steps(st):
        for b in range(2):
            i = st * 2 + b
            o = 1 - b

            @pl.when(i < _GNCH)
            def _():
                @pl.when(i + 1 < _GNCH)
                def _():
                    @pl.when(i >= 1)
                    def _():
                        pltpu.make_async_copy(
                            rbufs[o], g_hbm.at[pl.ds(ebase, _GCH)],
                            sems_o[o]).wait()

                    off = (i + 1) * _GCH
                    pltpu.async_copy(
                        xr_hbm.at[rowv.at[pl.ds(off, _GCH)]],
                        rbufs[o], sems_r[o])
                    pltpu.async_copy(
                        xc_hbm.at[colv.at[pl.ds(off, _GCH)]],
                        cbufs[o], sems_c[o])

                pltpu.make_async_copy(
                    xr_hbm.at[pl.ds(0, _GCH)], rbufs[b], sems_r[b]).wait()
                pltpu.make_async_copy(
                    xc_hbm.at[pl.ds(0, _GCH)], cbufs[b], sems_c[b]).wait()

                @pl.loop(0, _GCH)
                def _row_loop(j):
                    for k in range(HIDDEN_DIM // 16):
                        sl = pl.ds(k * 16, 16)
                        rbufs[b][j, sl] = rbufs[b][j, sl] + cbufs[b][j, sl]

                pltpu.async_copy(
                    rbufs[b], g_hbm.at[pl.ds(ebase + i * _GCH, _GCH)],
                    sems_o[b])

    pltpu.make_async_copy(
        rbuf1, g_hbm.at[pl.ds(ebase, _GCH)], sem_o1).wait()
    pltpu.make_async_copy(
        rbuf0, g_hbm.at[pl.ds(ebase, _GCH)], sem_o0).wait()


def _sc_gather(xr, xc, row, col):
    f = pl.kernel(
        _sc_gather_body,
        out_type=jax.ShapeDtypeStruct((N_EDGES, HIDDEN_DIM), jnp.float32),
        mesh=_mesh(),
        scratch_types=[
            pltpu.VMEM((_EPT,), jnp.int32),
            pltpu.VMEM((_EPT,), jnp.int32),
            pltpu.VMEM((_GCH, HIDDEN_DIM), jnp.float32),
            pltpu.VMEM((_GCH, HIDDEN_DIM), jnp.float32),
            pltpu.VMEM((_GCH, HIDDEN_DIM), jnp.float32),
            pltpu.VMEM((_GCH, HIDDEN_DIM), jnp.float32),
            pltpu.SemaphoreType.DMA,
            pltpu.SemaphoreType.DMA,
            pltpu.SemaphoreType.DMA,
            pltpu.SemaphoreType.DMA,
            pltpu.SemaphoreType.DMA,
            pltpu.SemaphoreType.DMA,
        ],
    )
    return f(xr, xc, row, col)


def _edge_mlp_body(g_ref, ea_ref, we1e_ref, be1_ref, we2_ref, be2_ref, out_ref):
    pre = (g_ref[...]
           + jnp.dot(ea_ref[...], we1e_ref[...],
                     preferred_element_type=jnp.float32)
           + be1_ref[...])
    h = jnp.maximum(pre, 0.0)
    out_ref[...] = (jnp.dot(h, we2_ref[...], preferred_element_type=jnp.float32)
                    + be2_ref[...])


def _edge_mlp(g, ea, we1e, be1, we2, be2):
    blk = 2000
    grid = (N_EDGES // blk,)
    return pl.pallas_call(
        _edge_mlp_body,
        grid=grid,
        in_specs=[
            pl.BlockSpec((blk, HIDDEN_DIM), lambda i: (i, 0)),
            pl.BlockSpec((blk, EDGE_DIM), lambda i: (i, 0)),
            pl.BlockSpec((EDGE_DIM, HIDDEN_DIM), lambda i: (0, 0)),
            pl.BlockSpec((1, HIDDEN_DIM), lambda i: (0, 0)),
            pl.BlockSpec((HIDDEN_DIM, EDGE_DIM), lambda i: (0, 0)),
            pl.BlockSpec((1, EDGE_DIM), lambda i: (0, 0)),
        ],
        out_specs=pl.BlockSpec((blk, EDGE_DIM), lambda i: (i, 0)),
        out_shape=jax.ShapeDtypeStruct((N_EDGES, EDGE_DIM), jnp.float32),
    )(g, ea, we1e, be1, we2, be2)


_ZCH = 400
_NZCH = N_NODES // _ZCH


_RCH = 80
_NRCH = N_NODES // _RCH


def _sc_scatter_body(e_hbm, col_hbm, out_hbm, cidx, ecompact, ebuf, nidx,
                     nbuf, acc, sem):
    c = lax.axis_index("c")
    s = lax.axis_index("s")
    w = s * _NC + c

    @pl.loop(0, _CHUNK)
    def _ze(j):
        for k in range(8):
            ebuf[j, pl.ds(k * 16, 16)] = jnp.zeros((16,), jnp.float32)

    @pl.loop(0, _RCH)
    def _zn(j):
        for k in range(8):
            nbuf[j, pl.ds(k * 16, 16)] = jnp.zeros((16,), jnp.float32)

    @pl.loop(0, 8)
    def _zero_chunks(i):
        zc = i * _NS + s

        @pl.when(zc < _NRCH)
        def _():
            @pl.loop(0, _RCH // 16)
            def _zi(k):
                nidx[pl.ds(k * 16, 16)] = (
                    zc * _RCH + k * 16 + lax.iota(jnp.int32, 16))

            pltpu.sync_copy(nbuf, acc.at[nidx])

    plsc.subcore_barrier()

    @pl.loop(0, _CPW)
    def _chunk_loop(i):
        cid = i * _NW + w

        @pl.when(cid < _NCHUNKS)
        def _():
            base = cid * _CHUNK
            pltpu.sync_copy(col_hbm.at[pl.ds(base, _CHUNK)], cidx)
            pltpu.sync_copy(e_hbm.at[pl.ds(base, _CHUNK)], ecompact)

            @pl.loop(0, _CHUNK)
            def _exp(j):
                ebuf[j, pl.ds(0, 16)] = ecompact[j, :]

            pltpu.sync_copy(ebuf, acc.at[cidx], add=True)

    plsc.subcore_barrier()

    @pl.loop(0, 8)
    def _out_chunks(i):
        zc = i * _NS + s

        @pl.when(zc < _NRCH)
        def _():
            @pl.loop(0, _RCH // 16)
            def _gi(k):
                nidx[pl.ds(k * 16, 16)] = (
                    zc * _RCH + k * 16 + lax.iota(jnp.int32, 16))

            pltpu.async_copy(acc.at[nidx], nbuf, sem).wait()
            pltpu.sync_copy(
                nbuf, out_hbm.at[pl.ds(c * N_NODES + zc * _RCH, _RCH)])


def _sc_scatter(e_new, col):
    f = pl.kernel(
        _sc_scatter_body,
        out_type=jax.ShapeDtypeStruct((_NC * N_NODES, HIDDEN_DIM),
                                      jnp.float32),
        mesh=_mesh(),
        scratch_types=[
            pltpu.VMEM((_CHUNK,), jnp.int32),
            pltpu.VMEM((_CHUNK, EDGE_DIM), jnp.float32),
            pltpu.VMEM((_CHUNK, HIDDEN_DIM), jnp.float32),
            pltpu.VMEM((_RCH,), jnp.int32),
            pltpu.VMEM((_RCH, HIDDEN_DIM), jnp.float32),
            pltpu.VMEM_SHARED((N_NODES, HIDDEN_DIM), jnp.float32),
            pltpu.SemaphoreType.DMA,
        ],
    )
    out = f(e_new, col)
    return out.reshape(_NC, N_NODES, HIDDEN_DIM)[:, :, :EDGE_DIM]


def _node_mlp_body(x_ref, m_ref, wn1x_ref, wn1m_ref, bn1_ref, wn2_ref,
                   bn2_ref, out_ref):
    m = m_ref[0] + m_ref[1]
    pre = (jnp.dot(x_ref[...], wn1x_ref[...],
                   preferred_element_type=jnp.float32)
           + jnp.dot(m, wn1m_ref[...], preferred_element_type=jnp.float32)
           + bn1_ref[...])
    h = jnp.maximum(pre, 0.0)
    out_ref[...] = (jnp.dot(h, wn2_ref[...], preferred_element_type=jnp.float32)
                    + bn2_ref[...])


def _node_mlp(x, partials, wn1x, wn1m, bn1, wn2, bn2):
    blk = 1000
    grid = (N_NODES // blk,)
    return pl.pallas_call(
        _node_mlp_body,
        grid=grid,
        in_specs=[
            pl.BlockSpec((blk, NODE_DIM), lambda i: (i, 0)),
            pl.BlockSpec((_NC, blk, EDGE_DIM), lambda i: (0, i, 0)),
            pl.BlockSpec((NODE_DIM, HIDDEN_DIM), lambda i: (0, 0)),
            pl.BlockSpec((EDGE_DIM, HIDDEN_DIM), lambda i: (0, 0)),
            pl.BlockSpec((1, HIDDEN_DIM), lambda i: (0, 0)),
            pl.BlockSpec((HIDDEN_DIM, NODE_DIM), lambda i: (0, 0)),
            pl.BlockSpec((1, NODE_DIM), lambda i: (0, 0)),
        ],
        out_specs=pl.BlockSpec((blk, NODE_DIM), lambda i: (i, 0)),
        out_shape=jax.ShapeDtypeStruct((N_NODES, NODE_DIM), jnp.float32),
    )(x, partials, wn1x, wn1m, bn1, wn2, bn2)


def kernel(x, edge_attr, W_e1, b_e1, W_e2, b_e2, W_n1, b_n1, W_n2, b_n2,
           edge_index):
    row = edge_index[0].astype(jnp.int32)
    col = edge_index[1].astype(jnp.int32)
    we1e = W_e1[:EDGE_DIM]
    wr = W_e1[EDGE_DIM:EDGE_DIM + NODE_DIM]
    wc = W_e1[EDGE_DIM + NODE_DIM:]
    wn1x = W_n1[:NODE_DIM]
    wn1m = W_n1[NODE_DIM:]

    xr, xc = _proj(x, wr, wc)
    g = _sc_gather(xr, xc, row, col)
    e_new = _edge_mlp(g, edge_attr, we1e, b_e1.reshape(1, -1),
                      W_e2, b_e2.reshape(1, -1))
    partials = _sc_scatter(e_new, col)
    x_new = _node_mlp(x, partials, wn1x, wn1m, b_n1.reshape(1, -1),
                      W_n2, b_n2.reshape(1, -1))
    return (x_new, e_new)


# speedup vs baseline: 3.8524x; 1.1462x over previous
import functools
import jax
import jax.numpy as jnp
from jax import lax
from jax.experimental import pallas as pl
from jax.experimental.pallas import tpu as pltpu
from jax.experimental.pallas import tpu_sc as plsc

N_NODES = 10000
N_EDGES = 320000
NODE_DIM = 128
EDGE_DIM = 16
HIDDEN_DIM = 128

_NC = 2
_NS = 16
_NW = _NC * _NS
_CHUNK = 128
_NCHUNKS = N_EDGES // _CHUNK
_CPW = -(-_NCHUNKS // _NW)

def _mesh():
    return plsc.VectorSubcoreMesh(core_axis_name="c", subcore_axis_name="s")


def _proj_body(x_ref, wr_ref, wc_ref, xr_ref, xc_ref):
    x = x_ref[...]
    xr_ref[...] = jnp.dot(x, wr_ref[...], preferred_element_type=jnp.float32)
    xc_ref[...] = jnp.dot(x, wc_ref[...], preferred_element_type=jnp.float32)


def _proj(x, wr, wc):
    blk = 1000
    grid = (N_NODES // blk,)
    return pl.pallas_call(
        _proj_body,
        grid=grid,
        in_specs=[
            pl.BlockSpec((blk, NODE_DIM), lambda i: (i, 0)),
            pl.BlockSpec((NODE_DIM, HIDDEN_DIM), lambda i: (0, 0)),
            pl.BlockSpec((NODE_DIM, HIDDEN_DIM), lambda i: (0, 0)),
        ],
        out_specs=[
            pl.BlockSpec((blk, HIDDEN_DIM), lambda i: (i, 0)),
            pl.BlockSpec((blk, HIDDEN_DIM), lambda i: (i, 0)),
        ],
        out_shape=[
            jax.ShapeDtypeStruct((N_NODES, HIDDEN_DIM), jnp.float32),
            jax.ShapeDtypeStruct((N_NODES, HIDDEN_DIM), jnp.float32),
        ],
    )(x, wr, wc)


_EPT = N_EDGES // _NW
_GCH = 80
_GNCH = _EPT // _GCH


def _sc_gather_body(xr_hbm, xc_hbm, row_hbm, col_hbm, g_hbm,
                    rowv, colv, rbuf0, rbuf1, cbuf0, cbuf1,
                    sem_r0, sem_r1, sem_c0, sem_c1, sem_o0, sem_o1):
    c = lax.axis_index("c")
    s = lax.axis_index("s")
    w = s * _NC + c
    ebase = w * _EPT
    rbufs = (rbuf0, rbuf1)
    cbufs = (cbuf0, cbuf1)
    sems_r = (sem_r0, sem_r1)
    sems_c = (sem_c0, sem_c1)
    sems_o = (sem_o0, sem_o1)

    pltpu.sync_copy(row_hbm.at[pl.ds(ebase, _EPT)], rowv)
    pltpu.sync_copy(col_hbm.at[pl.ds(ebase, _EPT)], colv)

    pltpu.async_copy(xr_hbm.at[rowv.at[pl.ds(0, _GCH)]], rbuf0, sem_r0)
    pltpu.async_copy(xc_hbm.at[colv.at[pl.ds(0, _GCH)]], cbuf0, sem_c0)

    @pl.loop(0, (_GNCH + 1) // 2)
    def _steps(st):
        for b in range(2):
            i = st * 2 + b
            o = 1 - b

            @pl.when(i < _GNCH)
            def _():
                @pl.when(i + 1 < _GNCH)
                def _():
                    @pl.when(i >= 1)
                    def _():
                        pltpu.make_async_copy(
                            rbufs[o], g_hbm.at[pl.ds(ebase, _GCH)],
                            sems_o[o]).wait()

                    off = (i + 1) * _GCH
                    pltpu.async_copy(
                        xr_hbm.at[rowv.at[pl.ds(off, _GCH)]],
                        rbufs[o], sems_r[o])
                    pltpu.async_copy(
                        xc_hbm.at[colv.at[pl.ds(off, _GCH)]],
                        cbufs[o], sems_c[o])

                pltpu.make_async_copy(
                    xr_hbm.at[pl.ds(0, _GCH)], rbufs[b], sems_r[b]).wait()
                pltpu.make_async_copy(
                    xc_hbm.at[pl.ds(0, _GCH)], cbufs[b], sems_c[b]).wait()

                @pl.loop(0, _GCH)
                def _row_loop(j):
                    for k in range(HIDDEN_DIM // 16):
                        sl = pl.ds(k * 16, 16)
                        rbufs[b][j, sl] = rbufs[b][j, sl] + cbufs[b][j, sl]

                pltpu.async_copy(
                    rbufs[b], g_hbm.at[pl.ds(ebase + i * _GCH, _GCH)],
                    sems_o[b])

    pltpu.make_async_copy(
        rbuf1, g_hbm.at[pl.ds(ebase, _GCH)], sem_o1).wait()
    pltpu.make_async_copy(
        rbuf0, g_hbm.at[pl.ds(ebase, _GCH)], sem_o0).wait()


def _sc_gather(xr, xc, row, col):
    f = pl.kernel(
        _sc_gather_body,
        out_type=jax.ShapeDtypeStruct((N_EDGES, HIDDEN_DIM), jnp.float32),
        mesh=_mesh(),
        scratch_types=[
            pltpu.VMEM((_EPT,), jnp.int32),
            pltpu.VMEM((_EPT,), jnp.int32),
            pltpu.VMEM((_GCH, HIDDEN_DIM), jnp.float32),
            pltpu.VMEM((_GCH, HIDDEN_DIM), jnp.float32),
            pltpu.VMEM((_GCH, HIDDEN_DIM), jnp.float32),
            pltpu.VMEM((_GCH, HIDDEN_DIM), jnp.float32),
            pltpu.SemaphoreType.DMA,
            pltpu.SemaphoreType.DMA,
            pltpu.SemaphoreType.DMA,
            pltpu.SemaphoreType.DMA,
            pltpu.SemaphoreType.DMA,
            pltpu.SemaphoreType.DMA,
        ],
    )
    return f(xr, xc, row, col)


def _edge_mlp_body(g_ref, ea_ref, we1e_ref, be1_ref, we2_ref, be2_ref, out_ref):
    pre = (g_ref[...]
           + jnp.dot(ea_ref[...], we1e_ref[...],
                     preferred_element_type=jnp.float32)
           + be1_ref[...])
    h = jnp.maximum(pre, 0.0)
    out_ref[...] = (jnp.dot(h, we2_ref[...], preferred_element_type=jnp.float32)
                    + be2_ref[...])


def _edge_mlp(g, ea, we1e, be1, we2, be2):
    blk = 2000
    grid = (N_EDGES // blk,)
    return pl.pallas_call(
        _edge_mlp_body,
        grid=grid,
        in_specs=[
            pl.BlockSpec((blk, HIDDEN_DIM), lambda i: (i, 0)),
            pl.BlockSpec((blk, EDGE_DIM), lambda i: (i, 0)),
            pl.BlockSpec((EDGE_DIM, HIDDEN_DIM), lambda i: (0, 0)),
            pl.BlockSpec((1, HIDDEN_DIM), lambda i: (0, 0)),
            pl.BlockSpec((HIDDEN_DIM, EDGE_DIM), lambda i: (0, 0)),
            pl.BlockSpec((1, EDGE_DIM), lambda i: (0, 0)),
        ],
        out_specs=pl.BlockSpec((blk, EDGE_DIM), lambda i: (i, 0)),
        out_shape=jax.ShapeDtypeStruct((N_EDGES, EDGE_DIM), jnp.float32),
    )(g, ea, we1e, be1, we2, be2)


_ZCH = 400
_NZCH = N_NODES // _ZCH


_RCH = 80
_NRCH = N_NODES // _RCH


def _sc_scatter_body(e_hbm, col_hbm, out_hbm, cidx, colv, ec0, ec1, ebuf,
                     nidx, nbuf, acc, sem, sem_e0, sem_e1):
    c = lax.axis_index("c")
    s = lax.axis_index("s")
    w = s * _NC + c


    @pl.loop(0, _RCH)
    def _zn(j):
        for k in range(8):
            nbuf[j, pl.ds(k * 16, 16)] = jnp.zeros((16,), jnp.float32)

    @pl.loop(0, _RCH // 16)
    def _zi(k):
        nidx[pl.ds(k * 16, 16)] = s * _RCH + k * 16 + lax.iota(jnp.int32, 16)

    pltpu.sync_copy(nbuf, acc.at[nidx])
    plsc.subcore_barrier()

    ebase = w * _EPT
    ecs = (ec0, ec1)
    sems_e = (sem_e0, sem_e1)
    pltpu.sync_copy(col_hbm.at[pl.ds(ebase, _EPT)], colv)
    pltpu.async_copy(e_hbm.at[pl.ds(ebase, _GCH)], ec0, sem_e0)

    @pl.loop(0, (_GNCH + 1) // 2)
    def _steps(st):
        for b in range(2):
            i = st * 2 + b
            o = 1 - b

            @pl.when(i < _GNCH)
            def _():
                @pl.when(i + 1 < _GNCH)
                def _():
                    pltpu.async_copy(
                        e_hbm.at[pl.ds(ebase + (i + 1) * _GCH, _GCH)],
                        ecs[o], sems_e[o])

                pltpu.make_async_copy(
                    e_hbm.at[pl.ds(0, _GCH)], ecs[b], sems_e[b]).wait()

                @pl.loop(0, _GCH // 16)
                def _ci(k):
                    cv = colv[pl.ds(i * _GCH + k * 16, 16)]
                    cidx[pl.ds(k * 16, 16)] = (
                        lax.shift_right_logical(cv, 3))

                @pl.loop(0, _GCH // 16)
                def _exp(g):
                    cv = colv[pl.ds(i * _GCH + g * 16, 16)]
                    slots = lax.rem(cv, 8)
                    for jj in range(16):
                        slot = slots[jj]
                        vals = ecs[b][g * 16 + jj, :]
                        for k in range(8):
                            z = jnp.where(slot == k, vals,
                                          jnp.zeros((16,), jnp.float32))
                            ebuf[g * 16 + jj, pl.ds(k * 16, 16)] = z

                pltpu.sync_copy(ebuf, acc.at[cidx], add=True)

    plsc.subcore_barrier()

    pltpu.async_copy(acc.at[nidx], nbuf, sem).wait()
    pltpu.sync_copy(nbuf, out_hbm.at[pl.ds((c * _NS + s) * _RCH, _RCH)])


def _sc_scatter(e_new, col):
    f = pl.kernel(
        _sc_scatter_body,
        out_type=jax.ShapeDtypeStruct((_NC * _NS * _RCH, HIDDEN_DIM),
                                      jnp.float32),
        mesh=_mesh(),
        scratch_types=[
            pltpu.VMEM((_GCH,), jnp.int32),
            pltpu.VMEM((_EPT,), jnp.int32),
            pltpu.VMEM((_GCH, EDGE_DIM), jnp.float32),
            pltpu.VMEM((_GCH, EDGE_DIM), jnp.float32),
            pltpu.VMEM((_GCH, HIDDEN_DIM), jnp.float32),
            pltpu.VMEM((_RCH,), jnp.int32),
            pltpu.VMEM((_RCH, HIDDEN_DIM), jnp.float32),
            pltpu.VMEM_SHARED((_NS * _RCH, HIDDEN_DIM), jnp.float32),
            pltpu.SemaphoreType.DMA,
            pltpu.SemaphoreType.DMA,
            pltpu.SemaphoreType.DMA,
        ],
    )
    out = f(e_new, col)
    out = out.reshape(_NC, _NS * _RCH, HIDDEN_DIM)[:, :N_NODES // 8]
    return out.reshape(_NC, N_NODES, EDGE_DIM)


def _node_mlp_body(x_ref, m_ref, wn1x_ref, wn1m_ref, bn1_ref, wn2_ref,
                   bn2_ref, out_ref):
    m = m_ref[0] + m_ref[1]
    pre = (jnp.dot(x_ref[...], wn1x_ref[...],
                   preferred_element_type=jnp.float32)
           + jnp.dot(m, wn1m_ref[...], preferred_element_type=jnp.float32)
           + bn1_ref[...])
    h = jnp.maximum(pre, 0.0)
    out_ref[...] = (jnp.dot(h, wn2_ref[...], preferred_element_type=jnp.float32)
                    + bn2_ref[...])


def _node_mlp(x, partials, wn1x, wn1m, bn1, wn2, bn2):
    blk = 1000
    grid = (N_NODES // blk,)
    return pl.pallas_call(
        _node_mlp_body,
        grid=grid,
        in_specs=[
            pl.BlockSpec((blk, NODE_DIM), lambda i: (i, 0)),
            pl.BlockSpec((_NC, blk, EDGE_DIM), lambda i: (0, i, 0)),
            pl.BlockSpec((NODE_DIM, HIDDEN_DIM), lambda i: (0, 0)),
            pl.BlockSpec((EDGE_DIM, HIDDEN_DIM), lambda i: (0, 0)),
            pl.BlockSpec((1, HIDDEN_DIM), lambda i: (0, 0)),
            pl.BlockSpec((HIDDEN_DIM, NODE_DIM), lambda i: (0, 0)),
            pl.BlockSpec((1, NODE_DIM), lambda i: (0, 0)),
        ],
        out_specs=pl.BlockSpec((blk, NODE_DIM), lambda i: (i, 0)),
        out_shape=jax.ShapeDtypeStruct((N_NODES, NODE_DIM), jnp.float32),
    )(x, partials, wn1x, wn1m, bn1, wn2, bn2)


def kernel(x, edge_attr, W_e1, b_e1, W_e2, b_e2, W_n1, b_n1, W_n2, b_n2,
           edge_index):
    row = edge_index[0].astype(jnp.int32)
    col = edge_index[1].astype(jnp.int32)
    we1e = W_e1[:EDGE_DIM]
    wr = W_e1[EDGE_DIM:EDGE_DIM + NODE_DIM]
    wc = W_e1[EDGE_DIM + NODE_DIM:]
    wn1x = W_n1[:NODE_DIM]
    wn1m = W_n1[NODE_DIM:]

    xr, xc = _proj(x, wr, wc)
    g = _sc_gather(xr, xc, row, col)
    e_new = _edge_mlp(g, edge_attr, we1e, b_e1.reshape(1, -1),
                      W_e2, b_e2.reshape(1, -1))
    partials = _sc_scatter(e_new, col)
    x_new = _node_mlp(x, partials, wn1x, wn1m, b_n1.reshape(1, -1),
                      W_n2, b_n2.reshape(1, -1))
    return (x_new, e_new)
